# Initial kernel scaffold; baseline (speedup 1.0000x reference)
import jax
import jax.numpy as jnp
from jax.experimental import pallas as pl


def kernel(cls_logit_0, center_logit_0, reg_logit_0, cls_logit_1, center_logit_1, reg_logit_1, cls_logit_2, center_logit_2, reg_logit_2, cls_logit_3, center_logit_3, reg_logit_3, cls_logit_4, center_logit_4, reg_logit_4, gt_box, labels):
    raise NotImplementedError("write your pallas kernel here")


# speedup vs baseline: 3.9350x; 3.9350x over previous
import functools

import numpy as np
import jax
import jax.numpy as jnp
from jax import lax
from jax.experimental import pallas as pl
from jax.experimental.pallas import tpu as pltpu
from jax.experimental.pallas import tpu_sc as plsc

_STRIDES = (8, 16, 32, 64, 128)
_LIMITS = ((-1.0, 64.0), (64.0, 128.0), (128.0, 256.0), (256.0, 512.0),
           (512.0, 999999.0))
_IMG = 1024
_B = 4
_M = 64
_HW = sum((_IMG // s) ** 2 for s in _STRIDES)
_NC = 2
_NS = 16
_NW = _NC * _NS
_WPB = _NW // _B
_CHUNK = 2752
_P = _WPB * _CHUNK
_NV = _CHUNK // 16


def _build_point_tables():
    xs, ys, lo, hi, rad = [], [], [], [], []
    for s, (l0, l1) in zip(_STRIDES, _LIMITS):
        h = _IMG // s
        c = np.arange(h, dtype=np.float32) * s + s // 2
        yy, xx = np.meshgrid(c, c, indexing="ij")
        n = h * h
        xs.append(xx.reshape(-1))
        ys.append(yy.reshape(-1))
        lo.append(np.full(n, l0, np.float32))
        hi.append(np.full(n, l1, np.float32))
        rad.append(np.full(n, s * 1.5, np.float32))
    pad = _P - _HW

    def cat(parts, padval):
        return np.concatenate(parts + [np.full(pad, padval, np.float32)])

    return (cat(xs, 0.0), cat(ys, 0.0), cat(lo, 0.0), cat(hi, -1.0),
            cat(rad, 1.0))


_XS, _YS, _LO, _HI, _RAD = _build_point_tables()


@functools.cache
def _build_sc_targets():
    mesh = plsc.VectorSubcoreMesh(core_axis_name="c", subcore_axis_name="s")
    return pl.kernel(
        _sc_targets_body,
        mesh=mesh,
        out_type=[
            jax.ShapeDtypeStruct((_B * _P,), jnp.int32),
            jax.ShapeDtypeStruct((_B * _P,), jnp.float32),
            jax.ShapeDtypeStruct((_B * _P,), jnp.float32),
            jax.ShapeDtypeStruct((_B * _P,), jnp.float32),
            jax.ShapeDtypeStruct((_B * _P,), jnp.float32),
            jax.ShapeDtypeStruct((_B * _P,), jnp.float32),
        ],
        scratch_types=[
            pltpu.VMEM((_CHUNK,), jnp.float32),
            pltpu.VMEM((_CHUNK,), jnp.float32),
            pltpu.VMEM((_CHUNK,), jnp.float32),
            pltpu.VMEM((_CHUNK,), jnp.float32),
            pltpu.VMEM((_CHUNK,), jnp.float32),
            pltpu.VMEM((_M * 4 * 16,), jnp.float32),
            pltpu.VMEM((_M * 16,), jnp.int32),
            pltpu.VMEM((_CHUNK,), jnp.int32),
            pltpu.VMEM((_CHUNK,), jnp.float32),
            pltpu.VMEM((_CHUNK,), jnp.float32),
            pltpu.VMEM((_CHUNK,), jnp.float32),
            pltpu.VMEM((_CHUNK,), jnp.float32),
            pltpu.VMEM((_CHUNK,), jnp.float32),
        ],
    )


def _sc_targets_body(xs_h, ys_h, lo_h, hi_h, rad_h, boxes_h, labs_h,
                cls_o, cen_o, l_o, t_o, r_o, b_o,
                xs_v, ys_v, lo_v, hi_v, rad_v, box_v, lab_v,
                cls_b, cen_b, l_b, t_b, r_b, b_b):
    w = lax.axis_index("s") * _NC + lax.axis_index("c")
    bat = w // _WPB
    base = (w % _WPB) * _CHUNK

    pltpu.sync_copy(xs_h.at[pl.ds(base, _CHUNK)], xs_v)
    pltpu.sync_copy(ys_h.at[pl.ds(base, _CHUNK)], ys_v)
    pltpu.sync_copy(lo_h.at[pl.ds(base, _CHUNK)], lo_v)
    pltpu.sync_copy(hi_h.at[pl.ds(base, _CHUNK)], hi_v)
    pltpu.sync_copy(rad_h.at[pl.ds(base, _CHUNK)], rad_v)
    pltpu.sync_copy(boxes_h.at[pl.ds(bat * (_M * 4 * 16), _M * 4 * 16)], box_v)
    pltpu.sync_copy(labs_h.at[pl.ds(bat * (_M * 16), _M * 16)], lab_v)

    def point_body(i, carry):
        o = i * 16
        xv = xs_v[pl.ds(o, 16)]
        yv = ys_v[pl.ds(o, 16)]
        lov = lo_v[pl.ds(o, 16)]
        hiv = hi_v[pl.ds(o, 16)]
        rv = rad_v[pl.ds(o, 16)]

        big = jnp.full((16,), 3.0e38, jnp.float32)
        one = jnp.full((16,), 1.0, jnp.float32)
        zero = jnp.full((16,), 0.0, jnp.float32)
        sentinel = jnp.full((16,), 99999999.0, jnp.float32)
        izero = jnp.full((16,), 0, jnp.int32)
        init = (big, one, one, one, one, izero, zero)

        def box_body(j, st):
            ba, bl, bt, br, bb, blab, pos = st
            q = j * 64
            x1 = box_v[pl.ds(q, 16)]
            y1 = box_v[pl.ds(q + 16, 16)]
            x2 = box_v[pl.ds(q + 32, 16)]
            y2 = box_v[pl.ds(q + 48, 16)]
            labj = lab_v[pl.ds(j * 16, 16)]
            lft = xv - x1
            top = yv - y1
            rgt = x2 - xv
            bot = y2 - yv
            area = (lft + rgt) * (top + bot)
            omin = jnp.minimum(jnp.minimum(lft, top), jnp.minimum(rgt, bot))
            omax = jnp.maximum(jnp.maximum(lft, top), jnp.maximum(rgt, bot))
            cx = (x1 + x2) * 0.5
            cy = (y1 + y2) * 0.5
            cdist = jnp.maximum(jnp.abs(xv - cx), jnp.abs(yv - cy))
            m = (omin > zero) & (omax > lov) & (omax <= hiv) & (cdist < rv)
            am = jnp.where(m, area, sentinel)
            upd = am < ba
            return (jnp.where(upd, am, ba),
                    jnp.where(upd, lft, bl),
                    jnp.where(upd, top, bt),
                    jnp.where(upd, rgt, br),
                    jnp.where(upd, bot, bb),
                    jnp.where(upd, labj, blab),
                    jnp.where(m, one, pos))

        ba, bl, bt, br, bb, blab, pos = lax.fori_loop(0, _M, box_body, init)
        posm = pos > jnp.full((16,), 0.5, jnp.float32)

        lrmin = jnp.minimum(bl, br)
        lrmax = jnp.maximum(bl, br)
        tbmin = jnp.minimum(bt, bb)
        tbmax = jnp.maximum(bt, bb)
        ratio = lrmin * tbmin / (lrmax * tbmax + 1e-10)
        rs = jnp.where(posm, ratio, one)
        bits = lax.bitcast_convert_type(rs, jnp.int32)
        s = lax.bitcast_convert_type(
            (bits >> 1) + jnp.full((16,), 0x1FBD1DF5, jnp.int32), jnp.float32)
        s = (s + rs / s) * 0.5
        s = (s + rs / s) * 0.5
        s = (s + rs / s) * 0.5

        neg1 = jnp.full((16,), -1.0, jnp.float32)
        cls_b[pl.ds(o, 16)] = jnp.where(posm, blab, izero)
        cen_b[pl.ds(o, 16)] = jnp.where(posm, s, neg1)
        l_b[pl.ds(o, 16)] = jnp.where(posm, bl, neg1)
        t_b[pl.ds(o, 16)] = jnp.where(posm, bt, neg1)
        r_b[pl.ds(o, 16)] = jnp.where(posm, br, neg1)
        b_b[pl.ds(o, 16)] = jnp.where(posm, bb, neg1)
        return carry

    lax.fori_loop(0, _NV, point_body, 0)

    off = bat * _P + base
    pltpu.sync_copy(cls_b, cls_o.at[pl.ds(off, _CHUNK)])
    pltpu.sync_copy(cen_b, cen_o.at[pl.ds(off, _CHUNK)])
    pltpu.sync_copy(l_b, l_o.at[pl.ds(off, _CHUNK)])
    pltpu.sync_copy(t_b, t_o.at[pl.ds(off, _CHUNK)])
    pltpu.sync_copy(r_b, r_o.at[pl.ds(off, _CHUNK)])
    pltpu.sync_copy(b_b, b_o.at[pl.ds(off, _CHUNK)])


def kernel(cls_logit_0, center_logit_0, reg_logit_0,
           cls_logit_1, center_logit_1, reg_logit_1,
           cls_logit_2, center_logit_2, reg_logit_2,
           cls_logit_3, center_logit_3, reg_logit_3,
           cls_logit_4, center_logit_4, reg_logit_4,
           gt_box, labels):
    boxes_bc = jnp.broadcast_to(
        gt_box.astype(jnp.float32)[..., None], (_B, _M, 4, 16)).reshape(-1)
    labs_bc = jnp.broadcast_to(
        labels.astype(jnp.int32)[..., None], (_B, _M, 16)).reshape(-1)
    cls_f, cen_f, l_f, t_f, r_f, b_f = _build_sc_targets()(
        jnp.asarray(_XS), jnp.asarray(_YS), jnp.asarray(_LO),
        jnp.asarray(_HI), jnp.asarray(_RAD), boxes_bc, labs_bc)
    cls_t = cls_f.reshape(_B, _P)[:, :_HW, None]
    cen_t = cen_f.reshape(_B, _P)[:, :_HW, None]
    reg_t = jnp.stack(
        [a.reshape(_B, _P)[:, :_HW] for a in (l_f, t_f, r_f, b_f)], axis=-1)
    return cls_t, cen_t, reg_t


# speedup vs baseline: 11.2612x; 2.8618x over previous
import functools

import jax
import jax.numpy as jnp
from jax import lax
from jax.experimental import pallas as pl
from jax.experimental.pallas import tpu as pltpu
from jax.experimental.pallas import tpu_sc as plsc

_STRIDES = (8, 16, 32, 64, 128)
_LIMITS = ((-1.0, 64.0), (64.0, 128.0), (128.0, 256.0), (256.0, 512.0),
           (512.0, 999999.0))
_IMG = 1024
_B = 4
_M = 64
_HS = tuple(_IMG // s for s in _STRIDES)
_HW = sum(h * h for h in _HS)
_LVL_BASE = tuple(sum(h * h for h in _HS[:i]) for i in range(5))
_NC = 2
_NS = 16
_NW = _NC * _NS
_WPB = _NW // _B
_CHUNK = 2752
_P = _WPB * _CHUNK
_NV = _CHUNK // 16
_SENTINEL = 99999999.0


@functools.cache
def _build_sc_targets():
    mesh = plsc.VectorSubcoreMesh(core_axis_name="c", subcore_axis_name="s")
    return pl.kernel(
        _sc_targets_body,
        mesh=mesh,
        compiler_params=pltpu.CompilerParams(needs_layout_passes=False),
        out_type=[
            jax.ShapeDtypeStruct((_B * _P,), jnp.int32),
            jax.ShapeDtypeStruct((_B * _P,), jnp.float32),
            jax.ShapeDtypeStruct((_B * _P,), jnp.float32),
            jax.ShapeDtypeStruct((_B * _P,), jnp.float32),
            jax.ShapeDtypeStruct((_B * _P,), jnp.float32),
            jax.ShapeDtypeStruct((_B * _P,), jnp.float32),
        ],
        scratch_types=[
            pltpu.VMEM((_M * 4 * 16,), jnp.float32),
            pltpu.VMEM((_M * 16,), jnp.int32),
            pltpu.VMEM((_CHUNK,), jnp.float32),
            pltpu.VMEM((_CHUNK,), jnp.int32),
            pltpu.VMEM((_CHUNK,), jnp.float32),
            pltpu.VMEM((_CHUNK,), jnp.float32),
            pltpu.VMEM((_CHUNK,), jnp.float32),
            pltpu.VMEM((_CHUNK,), jnp.float32),
        ],
    )


def _sc_targets_body(boxes_h, labs_h,
                     cls_o, cen_o, l_o, t_o, r_o, b_o,
                     box_v, lab_v, area_b, lab_b, l_b, t_b, r_b, b_b):
    w = lax.axis_index("s") * _NC + lax.axis_index("c")
    bat = w // _WPB
    base = (w % _WPB) * _CHUNK

    pltpu.sync_copy(boxes_h.at[pl.ds(bat * (_M * 4 * 16), _M * 4 * 16)], box_v)
    pltpu.sync_copy(labs_h.at[pl.ds(bat * (_M * 16), _M * 16)], lab_v)

    sentinel = jnp.full((16,), _SENTINEL, jnp.float32)
    zero = jnp.full((16,), 0.0, jnp.float32)
    one = jnp.full((16,), 1.0, jnp.float32)
    lane = lax.iota(jnp.int32, 16)
    dx = lane & 3
    dy = lane >> 2

    def init_body(i, carry):
        area_b[pl.ds(i * 16, 16)] = sentinel
        return carry

    lax.fori_loop(0, _NV, init_body, 0)

    for lv in range(5):
        s = float(_STRIDES[lv])
        inv_s = 1.0 / s
        h = _HS[lv]
        lo = float(_LIMITS[lv][0])
        hi = float(_LIMITS[lv][1])
        rad = 1.5 * s
        lvl_lo = _LVL_BASE[lv]
        lvl_hi = lvl_lo + h * h
        nb = lvl_lo - base

        def box_body(j, carry, inv_s=inv_s, s=s, h=h, lo=lo, hi=hi, rad=rad,
                     nb=nb):
            q = j * 64
            x1 = box_v[pl.ds(q, 16)]
            y1 = box_v[pl.ds(q + 16, 16)]
            x2 = box_v[pl.ds(q + 32, 16)]
            y2 = box_v[pl.ds(q + 48, 16)]
            labj = lab_v[pl.ds(j * 16, 16)]
            cx = (x1 + x2) * 0.5
            cy = (y1 + y2) * 0.5
            kx = (cx * inv_s - 0.5).astype(jnp.int32) - 1 + dx
            ky = (cy * inv_s - 0.5).astype(jnp.int32) - 1 + dy
            xw = (kx.astype(jnp.float32) + 0.5) * s
            yw = (ky.astype(jnp.float32) + 0.5) * s
            lft = xw - x1
            top = yw - y1
            rgt = x2 - xw
            bot = y2 - yw
            area = (lft + rgt) * (top + bot)
            omin = jnp.minimum(jnp.minimum(lft, top), jnp.minimum(rgt, bot))
            omax = jnp.maximum(jnp.maximum(lft, top), jnp.maximum(rgt, bot))
            cd = jnp.maximum(jnp.abs(xw - cx), jnp.abs(yw - cy))
            m = (omin > zero) & (omax > lo) & (omax <= hi) & (cd < rad)
            ploc = ky * h + kx + nb
            own = (ploc >= 0) & (ploc < _CHUNK)
            idx = jnp.clip(ploc, 0, _CHUNK - 1)
            cur = plsc.load_gather(area_b, [idx])
            upd = m & own & (area < cur)
            plsc.store_scatter(area_b, [idx], area, mask=upd)
            plsc.store_scatter(l_b, [idx], lft, mask=upd)
            plsc.store_scatter(t_b, [idx], top, mask=upd)
            plsc.store_scatter(r_b, [idx], rgt, mask=upd)
            plsc.store_scatter(b_b, [idx], bot, mask=upd)
            plsc.store_scatter(lab_b, [idx], labj, mask=upd)
            return carry

        @pl.when((lvl_hi > base) & (lvl_lo < base + _CHUNK))
        def _():
            lax.fori_loop(0, _M, box_body, 0)

    neg1 = jnp.full((16,), -1.0, jnp.float32)
    izero = jnp.full((16,), 0, jnp.int32)
    thresh = jnp.full((16,), 9.0e7, jnp.float32)
    magic = jnp.full((16,), 0x1FBD1DF5, jnp.int32)

    def fin_body(i, carry):
        o = i * 16
        av = area_b[pl.ds(o, 16)]
        bl = l_b[pl.ds(o, 16)]
        bt = t_b[pl.ds(o, 16)]
        br = r_b[pl.ds(o, 16)]
        bb = b_b[pl.ds(o, 16)]
        blab = lab_b[pl.ds(o, 16)]
        posm = av < thresh
        lrmin = jnp.minimum(bl, br)
        lrmax = jnp.maximum(bl, br)
        tbmin = jnp.minimum(bt, bb)
        tbmax = jnp.maximum(bt, bb)
        ratio = lrmin * tbmin / (lrmax * tbmax + 1e-10)
        rs = jnp.where(posm, ratio, one)
        sq = lax.bitcast_convert_type(
            (lax.bitcast_convert_type(rs, jnp.int32) >> 1) + magic,
            jnp.float32)
        sq = (sq + rs / sq) * 0.5
        sq = (sq + rs / sq) * 0.5
        sq = (sq + rs / sq) * 0.5
        area_b[pl.ds(o, 16)] = jnp.where(posm, sq, neg1)
        lab_b[pl.ds(o, 16)] = jnp.where(posm, blab, izero)
        l_b[pl.ds(o, 16)] = jnp.where(posm, bl, neg1)
        t_b[pl.ds(o, 16)] = jnp.where(posm, bt, neg1)
        r_b[pl.ds(o, 16)] = jnp.where(posm, br, neg1)
        b_b[pl.ds(o, 16)] = jnp.where(posm, bb, neg1)
        return carry

    lax.fori_loop(0, _NV, fin_body, 0)

    off = bat * _P + base
    pltpu.sync_copy(lab_b, cls_o.at[pl.ds(off, _CHUNK)])
    pltpu.sync_copy(area_b, cen_o.at[pl.ds(off, _CHUNK)])
    pltpu.sync_copy(l_b, l_o.at[pl.ds(off, _CHUNK)])
    pltpu.sync_copy(t_b, t_o.at[pl.ds(off, _CHUNK)])
    pltpu.sync_copy(r_b, r_o.at[pl.ds(off, _CHUNK)])
    pltpu.sync_copy(b_b, b_o.at[pl.ds(off, _CHUNK)])


def kernel(cls_logit_0, center_logit_0, reg_logit_0,
           cls_logit_1, center_logit_1, reg_logit_1,
           cls_logit_2, center_logit_2, reg_logit_2,
           cls_logit_3, center_logit_3, reg_logit_3,
           cls_logit_4, center_logit_4, reg_logit_4,
           gt_box, labels):
    boxes_bc = jnp.broadcast_to(
        gt_box.astype(jnp.float32)[..., None], (_B, _M, 4, 16)).reshape(-1)
    labs_bc = jnp.broadcast_to(
        labels.astype(jnp.int32)[..., None], (_B, _M, 16)).reshape(-1)
    cls_f, cen_f, l_f, t_f, r_f, b_f = _build_sc_targets()(boxes_bc, labs_bc)
    cls_t = cls_f.reshape(_B, _P)[:, :_HW, None]
    cen_t = cen_f.reshape(_B, _P)[:, :_HW, None]
    reg_t = jnp.stack(
        [a.reshape(_B, _P)[:, :_HW] for a in (l_f, t_f, r_f, b_f)], axis=-1)
    return cls_t, cen_t, reg_t
